# baseline (speedup 1.0000x reference)
import jax
import jax.numpy as jnp
from jax.experimental import pallas as pl
from jax.experimental.pallas import tpu as pltpu

K = 16
TN = 128


def _softmax_k(lg):
    m = jnp.max(lg, axis=1, keepdims=True)
    e = jnp.exp(lg - m)
    return e * pl.reciprocal(jnp.sum(e, axis=1, keepdims=True), approx=True)


def _fused_kernel(x_ref, xq_ref, g_ref,
                  w_if_ref, sc_if_ref, sh_if_ref,
                  w_c_ref, w_nd_ref, sc_lse_ref, sh_lse_ref,
                  w_att1_ref, w_fc1_ref, b1_ref,
                  w_p1_ref, sc_p1_ref, sh_p1_ref,
                  w_att2_ref, w_fc2_ref, b2_ref,
                  w_p2_ref, sc_p2_ref, sh_p2_ref,
                  w_mo_ref, sc_mo_ref, sh_mo_ref,
                  o_ref):
    tn = x_ref.shape[0]

    y = jnp.dot(x_ref[...], w_if_ref[...], preferred_element_type=jnp.float32)
    y = jnp.maximum(y * sc_if_ref[...] + sh_if_ref[...], 0.0)
    residual = y[:, :256]
    fc1 = y[:, 256:384]

    g = g_ref[0]
    cproj = jnp.dot(xq_ref[...], w_c_ref[...],
                    preferred_element_type=jnp.float32)
    nd = jnp.dot(g.reshape(K * tn, 4), w_nd_ref[...],
                 preferred_element_type=jnp.float32)
    lse = nd.reshape(K, tn, 256) + cproj[None, :, :]
    lse = jnp.maximum(lse * sc_lse_ref[0][None, None, :]
                      + sh_lse_ref[0][None, None, :], 0.0)
    lsef = lse.reshape(K * tn, 256)
    lse1f = lsef[:, :128]
    lse2f = lsef[:, 128:]

    def attend(lse_flat, fc, w_att_ref, w_fc_ref, b_ref,
               w_p_ref, sc_p_ref, sh_p_ref):
        f = jnp.dot(fc, w_fc_ref[...],
                    preferred_element_type=jnp.float32) + b_ref[...]
        lg = jnp.dot(lse_flat, w_att_ref[...],
                     preferred_element_type=jnp.float32)
        lg = lg.reshape(K, tn, 256) + f[None, :, :]
        m = jnp.max(lg, axis=0, keepdims=True)
        e = jnp.exp(lg - m)
        s = e * pl.reciprocal(jnp.sum(e, axis=0, keepdims=True), approx=True)
        agg_a = jnp.sum(s[:, :, :128] * lse_flat.reshape(K, tn, 128), axis=0)
        agg_b = fc * jnp.sum(s[:, :, 128:], axis=0)
        p = jnp.dot(jnp.concatenate([agg_a, agg_b], axis=1), w_p_ref[...],
                    preferred_element_type=jnp.float32)
        return jnp.maximum(p * sc_p_ref[...] + sh_p_ref[...], 0.0)

    att1 = attend(lse1f, fc1, w_att1_ref, w_fc1_ref, b1_ref,
                  w_p1_ref, sc_p1_ref, sh_p1_ref)
    att2 = attend(lse2f, att1, w_att2_ref, w_fc2_ref, b2_ref,
                  w_p2_ref, sc_p2_ref, sh_p2_ref)

    z = jnp.dot(att2, w_mo_ref[...], preferred_element_type=jnp.float32)
    z = jnp.maximum(z * sc_mo_ref[...] + sh_mo_ref[...], 0.0) + residual
    o_ref[...] = jnp.where(z >= 0.0, z, 0.01 * z)


def _const_spec(shape):
    return pl.BlockSpec(shape, lambda *_: (0,) * len(shape))


QT = 256


def _knn_kernel(xq_ref, pts_ref, o_ref):
    xq = xq_ref[0]
    px = pts_ref[0, 0:1, :]
    py = pts_ref[0, 1:2, :]
    pz = pts_ref[0, 2:3, :]
    dx = xq[:, 0:1] - px
    dy = xq[:, 1:2] - py
    dz = xq[:, 2:3] - pz
    d2 = dx * dx + dy * dy + dz * dz
    lane = jax.lax.broadcasted_iota(jnp.int32, d2.shape, 1)
    key = ((pltpu.bitcast(d2, jnp.int32) & ~jnp.int32(0xFFF)) | lane) \
        + jnp.int32(0x20000000)
    work = pltpu.bitcast(key, jnp.float32)
    bound = jnp.full((xq.shape[0], 1), -1.0, jnp.float32)
    for k in range(K):
        cand = jnp.where(work > bound, work, jnp.inf)
        m = jnp.min(cand, axis=1, keepdims=True)
        bound = m
        eq = work == m
        nx = jnp.sum(jnp.where(eq, px, 0.0), axis=1, keepdims=True)
        ny = jnp.sum(jnp.where(eq, py, 0.0), axis=1, keepdims=True)
        nz = jnp.sum(jnp.where(eq, pz, 0.0), axis=1, keepdims=True)
        de = jnp.sum(jnp.where(eq, d2, 0.0), axis=1, keepdims=True)
        o_ref[0, k] = jnp.concatenate([nx, ny, nz, de], axis=1)


def _knn(xyz, xq):
    B, N, _ = xyz.shape
    qt = min(QT, N)
    pts = jnp.transpose(xq, (0, 2, 1))
    g = pl.pallas_call(
        _knn_kernel,
        out_shape=jax.ShapeDtypeStruct((B, K, N, 4), jnp.float32),
        grid=(B, N // qt),
        in_specs=[
            pl.BlockSpec((1, qt, 8), lambda b, i: (b, i, 0)),
            pl.BlockSpec((1, 8, N), lambda b, i: (b, 0, 0)),
        ],
        out_specs=pl.BlockSpec((1, K, qt, 4), lambda b, i: (b, 0, i, 0)),
        compiler_params=pltpu.CompilerParams(
            dimension_semantics=("parallel", "parallel"),
            vmem_limit_bytes=100 * 1024 * 1024,
        ),
    )(xq, pts)
    return g


def kernel(feat, xyz,
           in_fused_w, in_fused_sc, in_fused_sh,
           mlp_out_w, mlp_out_sc, mlp_out_sh,
           lse1_w, lse1_sc, lse1_sh,
           lse2_w, lse2_sc, lse2_sh,
           att1_w_aa, att1_w_ab, att1_w_ba, att1_w_bb, att1_b_a, att1_b_b,
           att2_w_aa, att2_w_ab, att2_w_ba, att2_w_bb, att2_b_a, att2_b_b,
           pool1_w_a, pool1_w_b, pool1_sc, pool1_sh,
           pool2_w_a, pool2_w_b, pool2_sc, pool2_sh):
    B, d_in, N, _ = feat.shape
    BN = B * N

    x = jnp.transpose(feat[..., 0], (0, 2, 1)).reshape(BN, d_in)

    xq_b = jnp.concatenate(
        [xyz, jnp.zeros((B, N, 5), jnp.float32)], axis=-1)
    g = _knn(xyz, xq_b)
    xq = xq_b.reshape(BN, 8)

    def _parts(w):
        zc = jnp.zeros((5, w.shape[1]), w.dtype)
        w_c = jnp.concatenate([w[0:3] + w[6:9], zc], axis=0)
        w_nd = jnp.concatenate([w[3:6] - w[6:9], w[9:10]], axis=0)
        return w_c, w_nd
    c1, nd1 = _parts(lse1_w)
    c2, nd2 = _parts(lse2_w)
    w_c = jnp.concatenate([c1, c2], axis=1)
    w_nd = jnp.concatenate([nd1, nd2], axis=1)
    sc_lse = jnp.concatenate([lse1_sc, lse2_sc], axis=1)
    sh_lse = jnp.concatenate([lse1_sh, lse2_sh], axis=1)
    w_att1 = jnp.concatenate([att1_w_aa, att1_w_ab], axis=1)
    w_fc1 = jnp.concatenate([att1_w_ba, att1_w_bb], axis=1)
    b1 = jnp.concatenate([att1_b_a, att1_b_b], axis=1)
    w_att2 = jnp.concatenate([att2_w_aa, att2_w_ab], axis=1)
    w_fc2 = jnp.concatenate([att2_w_ba, att2_w_bb], axis=1)
    b2 = jnp.concatenate([att2_b_a, att2_b_b], axis=1)
    w_p1 = jnp.concatenate([pool1_w_a, pool1_w_b], axis=0)
    w_p2 = jnp.concatenate([pool2_w_a, pool2_w_b], axis=0)

    nb = N // TN
    out = pl.pallas_call(
        _fused_kernel,
        out_shape=jax.ShapeDtypeStruct((BN, 256), jnp.float32),
        grid=(BN // TN,),
        in_specs=[
            pl.BlockSpec((TN, 128), lambda i: (i, 0)),
            pl.BlockSpec((TN, 8), lambda i: (i, 0)),
            pl.BlockSpec((1, K, TN, 4), lambda i: (i // nb, 0, i % nb, 0)),
            _const_spec((128, 384)), _const_spec((1, 384)), _const_spec((1, 384)),
            _const_spec((8, 256)), _const_spec((4, 256)),
            _const_spec((1, 256)), _const_spec((1, 256)),
            _const_spec((128, 256)), _const_spec((128, 256)), _const_spec((1, 256)),
            _const_spec((256, 128)), _const_spec((1, 128)), _const_spec((1, 128)),
            _const_spec((128, 256)), _const_spec((128, 256)), _const_spec((1, 256)),
            _const_spec((256, 128)), _const_spec((1, 128)), _const_spec((1, 128)),
            _const_spec((128, 256)), _const_spec((1, 256)), _const_spec((1, 256)),
        ],
        out_specs=pl.BlockSpec((TN, 256), lambda i: (i, 0)),
        compiler_params=pltpu.CompilerParams(
            dimension_semantics=("parallel",),
            vmem_limit_bytes=100 * 1024 * 1024,
        ),
    )(x, xq, g,
      in_fused_w, in_fused_sc, in_fused_sh,
      w_c, w_nd, sc_lse, sh_lse,
      w_att1, w_fc1, b1, w_p1, pool1_sc, pool1_sh,
      w_att2, w_fc2, b2, w_p2, pool2_sc, pool2_sh,
      mlp_out_w, mlp_out_sc, mlp_out_sh)

    out = out.reshape(B, N, 256)
    return jnp.transpose(out, (0, 2, 1))[..., None]


# speedup vs baseline: 1.4268x; 1.4268x over previous
import jax
import jax.numpy as jnp
from jax.experimental import pallas as pl
from jax.experimental.pallas import tpu as pltpu

K = 16
TN = 128


def _softmax_k(lg):
    m = jnp.max(lg, axis=1, keepdims=True)
    e = jnp.exp(lg - m)
    return e * pl.reciprocal(jnp.sum(e, axis=1, keepdims=True), approx=True)


def _fused_kernel(x_ref, xq_ref, g_ref,
                  w_if_ref, sc_if_ref, sh_if_ref,
                  w_c_ref, w_nd_ref, sc_lse_ref, sh_lse_ref,
                  w_att1_ref, w_fc1_ref, b1_ref,
                  w_p1_ref, sc_p1_ref, sh_p1_ref,
                  w_att2_ref, w_fc2_ref, b2_ref,
                  w_p2_ref, sc_p2_ref, sh_p2_ref,
                  w_mo_ref, sc_mo_ref, sh_mo_ref,
                  o_ref):
    tn = x_ref.shape[0]

    y = jnp.dot(x_ref[...], w_if_ref[...], preferred_element_type=jnp.float32)
    y = jnp.maximum(y * sc_if_ref[...] + sh_if_ref[...], 0.0)
    residual = y[:, :256]
    fc1 = y[:, 256:384]

    g = g_ref[0]
    cproj = jnp.dot(xq_ref[...], w_c_ref[...],
                    preferred_element_type=jnp.float32)
    nd = jnp.dot(g.reshape(K * tn, 4), w_nd_ref[...],
                 preferred_element_type=jnp.float32)
    lse = nd.reshape(K, tn, 256) + cproj[None, :, :]
    lse = jnp.maximum(lse * sc_lse_ref[0][None, None, :]
                      + sh_lse_ref[0][None, None, :], 0.0)
    lsef = lse.reshape(K * tn, 256)
    lse1f = lsef[:, :128]
    lse2f = lsef[:, 128:]

    def attend(lse_flat, fc, w_att_ref, w_fc_ref, b_ref,
               w_p_ref, sc_p_ref, sh_p_ref):
        f = jnp.dot(fc, w_fc_ref[...],
                    preferred_element_type=jnp.float32) + b_ref[...]
        lg = jnp.dot(lse_flat, w_att_ref[...],
                     preferred_element_type=jnp.float32)
        lg = lg.reshape(K, tn, 256) + f[None, :, :]
        m = jnp.max(lg, axis=0, keepdims=True)
        e = jnp.exp(lg - m)
        s = e * pl.reciprocal(jnp.sum(e, axis=0, keepdims=True), approx=True)
        agg_a = jnp.sum(s[:, :, :128] * lse_flat.reshape(K, tn, 128), axis=0)
        agg_b = fc * jnp.sum(s[:, :, 128:], axis=0)
        p = jnp.dot(jnp.concatenate([agg_a, agg_b], axis=1), w_p_ref[...],
                    preferred_element_type=jnp.float32)
        return jnp.maximum(p * sc_p_ref[...] + sh_p_ref[...], 0.0)

    att1 = attend(lse1f, fc1, w_att1_ref, w_fc1_ref, b1_ref,
                  w_p1_ref, sc_p1_ref, sh_p1_ref)
    att2 = attend(lse2f, att1, w_att2_ref, w_fc2_ref, b2_ref,
                  w_p2_ref, sc_p2_ref, sh_p2_ref)

    z = jnp.dot(att2, w_mo_ref[...], preferred_element_type=jnp.float32)
    z = jnp.maximum(z * sc_mo_ref[...] + sh_mo_ref[...], 0.0) + residual
    o_ref[...] = jnp.where(z >= 0.0, z, 0.01 * z)


def _const_spec(shape):
    return pl.BlockSpec(shape, lambda *_: (0,) * len(shape))


QT = 256


def _knn_kernel(xq_ref, pts_ref, o_ref):
    xq = xq_ref[0]
    px = pts_ref[0, 0:1, :]
    py = pts_ref[0, 1:2, :]
    pz = pts_ref[0, 2:3, :]
    dx = xq[:, 0:1] - px
    dy = xq[:, 1:2] - py
    dz = xq[:, 2:3] - pz
    d2 = dx * dx + dy * dy + dz * dz
    lane = jax.lax.broadcasted_iota(jnp.int32, d2.shape, 1)
    key = ((pltpu.bitcast(d2, jnp.int32) & ~jnp.int32(0xFFF)) | lane) \
        + jnp.int32(0x20000000)
    work = pltpu.bitcast(key, jnp.float32)
    bound = jnp.full((xq.shape[0], 1), -1.0, jnp.float32)
    cols = []
    for _ in range(K):
        cand = jnp.where(work > bound, work, jnp.inf)
        m = jnp.min(cand, axis=1, keepdims=True)
        cols.append(m)
        bound = m
    packed = jnp.concatenate(cols, axis=1)
    o_ref[0] = pltpu.bitcast(packed, jnp.int32) & jnp.int32(0xFFF)


def _knn(xyz, xq):
    B, N, _ = xyz.shape
    qt = min(QT, N)
    pts = jnp.transpose(xq, (0, 2, 1))
    idx = pl.pallas_call(
        _knn_kernel,
        out_shape=jax.ShapeDtypeStruct((B, N, K), jnp.int32),
        grid=(B, N // qt),
        in_specs=[
            pl.BlockSpec((1, qt, 8), lambda b, i: (b, i, 0)),
            pl.BlockSpec((1, 8, N), lambda b, i: (b, 0, 0)),
        ],
        out_specs=pl.BlockSpec((1, qt, K), lambda b, i: (b, i, 0)),
        compiler_params=pltpu.CompilerParams(
            dimension_semantics=("parallel", "parallel"),
            vmem_limit_bytes=100 * 1024 * 1024,
        ),
    )(xq, pts)
    return idx


def kernel(feat, xyz,
           in_fused_w, in_fused_sc, in_fused_sh,
           mlp_out_w, mlp_out_sc, mlp_out_sh,
           lse1_w, lse1_sc, lse1_sh,
           lse2_w, lse2_sc, lse2_sh,
           att1_w_aa, att1_w_ab, att1_w_ba, att1_w_bb, att1_b_a, att1_b_b,
           att2_w_aa, att2_w_ab, att2_w_ba, att2_w_bb, att2_b_a, att2_b_b,
           pool1_w_a, pool1_w_b, pool1_sc, pool1_sh,
           pool2_w_a, pool2_w_b, pool2_sc, pool2_sh):
    B, d_in, N, _ = feat.shape
    BN = B * N

    x = jnp.transpose(feat[..., 0], (0, 2, 1)).reshape(BN, d_in)

    xq_b = jnp.concatenate(
        [xyz, jnp.zeros((B, N, 5), jnp.float32)], axis=-1)
    idx = _knn(xyz, xq_b)
    xq = xq_b.reshape(BN, 8)

    gidx = (idx + (jnp.arange(B, dtype=jnp.int32) * N)[:, None, None]).reshape(-1)
    nxyz = jnp.take(xyz.reshape(BN, 3), gidx, axis=0).reshape(B, N, K, 3)
    d2e = jnp.sum((xyz[:, :, None, :] - nxyz) ** 2, axis=-1)
    g4 = jnp.concatenate([nxyz, d2e[..., None]], axis=-1)
    g = jnp.transpose(g4, (0, 2, 1, 3))

    def _parts(w):
        zc = jnp.zeros((5, w.shape[1]), w.dtype)
        w_c = jnp.concatenate([w[0:3] + w[6:9], zc], axis=0)
        w_nd = jnp.concatenate([w[3:6] - w[6:9], w[9:10]], axis=0)
        return w_c, w_nd
    c1, nd1 = _parts(lse1_w)
    c2, nd2 = _parts(lse2_w)
    w_c = jnp.concatenate([c1, c2], axis=1)
    w_nd = jnp.concatenate([nd1, nd2], axis=1)
    sc_lse = jnp.concatenate([lse1_sc, lse2_sc], axis=1)
    sh_lse = jnp.concatenate([lse1_sh, lse2_sh], axis=1)
    w_att1 = jnp.concatenate([att1_w_aa, att1_w_ab], axis=1)
    w_fc1 = jnp.concatenate([att1_w_ba, att1_w_bb], axis=1)
    b1 = jnp.concatenate([att1_b_a, att1_b_b], axis=1)
    w_att2 = jnp.concatenate([att2_w_aa, att2_w_ab], axis=1)
    w_fc2 = jnp.concatenate([att2_w_ba, att2_w_bb], axis=1)
    b2 = jnp.concatenate([att2_b_a, att2_b_b], axis=1)
    w_p1 = jnp.concatenate([pool1_w_a, pool1_w_b], axis=0)
    w_p2 = jnp.concatenate([pool2_w_a, pool2_w_b], axis=0)

    nb = N // TN
    out = pl.pallas_call(
        _fused_kernel,
        out_shape=jax.ShapeDtypeStruct((BN, 256), jnp.float32),
        grid=(BN // TN,),
        in_specs=[
            pl.BlockSpec((TN, 128), lambda i: (i, 0)),
            pl.BlockSpec((TN, 8), lambda i: (i, 0)),
            pl.BlockSpec((1, K, TN, 4), lambda i: (i // nb, 0, i % nb, 0)),
            _const_spec((128, 384)), _const_spec((1, 384)), _const_spec((1, 384)),
            _const_spec((8, 256)), _const_spec((4, 256)),
            _const_spec((1, 256)), _const_spec((1, 256)),
            _const_spec((128, 256)), _const_spec((128, 256)), _const_spec((1, 256)),
            _const_spec((256, 128)), _const_spec((1, 128)), _const_spec((1, 128)),
            _const_spec((128, 256)), _const_spec((128, 256)), _const_spec((1, 256)),
            _const_spec((256, 128)), _const_spec((1, 128)), _const_spec((1, 128)),
            _const_spec((128, 256)), _const_spec((1, 256)), _const_spec((1, 256)),
        ],
        out_specs=pl.BlockSpec((TN, 256), lambda i: (i, 0)),
        compiler_params=pltpu.CompilerParams(
            dimension_semantics=("parallel",),
            vmem_limit_bytes=100 * 1024 * 1024,
        ),
    )(x, xq, g,
      in_fused_w, in_fused_sc, in_fused_sh,
      w_c, w_nd, sc_lse, sh_lse,
      w_att1, w_fc1, b1, w_p1, pool1_sc, pool1_sh,
      w_att2, w_fc2, b2, w_p2, pool2_sc, pool2_sh,
      mlp_out_w, mlp_out_sc, mlp_out_sh)

    out = out.reshape(B, N, 256)
    return jnp.transpose(out, (0, 2, 1))[..., None]


# speedup vs baseline: 2.3437x; 1.6426x over previous
import jax
import jax.numpy as jnp
from jax.experimental import pallas as pl
from jax.experimental.pallas import tpu as pltpu

K = 16
TN = 128


def _softmax_k(lg):
    m = jnp.max(lg, axis=1, keepdims=True)
    e = jnp.exp(lg - m)
    return e * pl.reciprocal(jnp.sum(e, axis=1, keepdims=True), approx=True)


def _fused_kernel(x_ref, xq_ref, idx_ref, ptsb_ref,
                  w_if_ref, sc_if_ref, sh_if_ref,
                  w_c_ref, w_n_ref, w_d_ref, sc_lse_ref, sh_lse_ref,
                  w_att1_ref, w_fc1_ref, b1_ref,
                  w_p1_ref, sc_p1_ref, sh_p1_ref,
                  w_att2_ref, w_fc2_ref, b2_ref,
                  w_p2_ref, sc_p2_ref, sh_p2_ref,
                  w_mo_ref, sc_mo_ref, sh_mo_ref,
                  o_ref, gath_ref):
    tn = x_ref.shape[0]

    def body(i, _):
        gath_ref[i, :] = ptsb_ref[0, idx_ref[i], :]
        return 0
    jax.lax.fori_loop(0, tn * K, body, 0, unroll=16)

    y = jnp.dot(x_ref[...], w_if_ref[...], preferred_element_type=jnp.float32)
    y = jnp.maximum(y * sc_if_ref[...] + sh_if_ref[...], 0.0)
    residual = y[:, :256]
    fc1 = y[:, 256:384]

    g = gath_ref[...]
    xq = xq_ref[...]
    diff = g.reshape(tn, K, 8) - xq[:, None, :]
    d2 = jnp.sum(diff * diff, axis=2, keepdims=True)
    cproj = jnp.dot(xq, w_c_ref[...], preferred_element_type=jnp.float32)
    nproj = jnp.dot(g, w_n_ref[...], preferred_element_type=jnp.float32)
    lse = nproj.reshape(tn, K, 256) + cproj[:, None, :] + d2 * w_d_ref[0][None, None, :]
    lse = jnp.maximum(lse * sc_lse_ref[0][None, None, :]
                      + sh_lse_ref[0][None, None, :], 0.0)
    lsef = lse.reshape(tn * K, 256)
    lse1f = lsef[:, :128]
    lse2f = lsef[:, 128:]

    def attend(lse_flat, fc, w_att_ref, w_fc_ref, b_ref,
               w_p_ref, sc_p_ref, sh_p_ref):
        f = jnp.dot(fc, w_fc_ref[...],
                    preferred_element_type=jnp.float32) + b_ref[...]
        lg = jnp.dot(lse_flat, w_att_ref[...],
                     preferred_element_type=jnp.float32)
        lg = lg.reshape(tn, K, 256) + f[:, None, :]
        m = jnp.max(lg, axis=1, keepdims=True)
        e = jnp.exp(lg - m)
        s = e * pl.reciprocal(jnp.sum(e, axis=1, keepdims=True), approx=True)
        agg_a = jnp.sum(s[:, :, :128] * lse_flat.reshape(tn, K, 128), axis=1)
        agg_b = fc * jnp.sum(s[:, :, 128:], axis=1)
        p = jnp.dot(jnp.concatenate([agg_a, agg_b], axis=1), w_p_ref[...],
                    preferred_element_type=jnp.float32)
        return jnp.maximum(p * sc_p_ref[...] + sh_p_ref[...], 0.0)

    att1 = attend(lse1f, fc1, w_att1_ref, w_fc1_ref, b1_ref,
                  w_p1_ref, sc_p1_ref, sh_p1_ref)
    att2 = attend(lse2f, att1, w_att2_ref, w_fc2_ref, b2_ref,
                  w_p2_ref, sc_p2_ref, sh_p2_ref)

    z = jnp.dot(att2, w_mo_ref[...], preferred_element_type=jnp.float32)
    z = jnp.maximum(z * sc_mo_ref[...] + sh_mo_ref[...], 0.0) + residual
    o_ref[...] = jnp.where(z >= 0.0, z, 0.01 * z)


def _const_spec(shape):
    return pl.BlockSpec(shape, lambda *_: (0,) * len(shape))


QT = 256


def _knn_kernel(xq_ref, pts_ref, o_ref):
    xq = xq_ref[0]
    px = pts_ref[0, 0:1, :]
    py = pts_ref[0, 1:2, :]
    pz = pts_ref[0, 2:3, :]
    dx = xq[:, 0:1] - px
    dy = xq[:, 1:2] - py
    dz = xq[:, 2:3] - pz
    d2 = dx * dx + dy * dy + dz * dz
    lane = jax.lax.broadcasted_iota(jnp.int32, d2.shape, 1)
    key = ((pltpu.bitcast(d2, jnp.int32) & ~jnp.int32(0xFFF)) | lane) \
        + jnp.int32(0x20000000)
    work = pltpu.bitcast(key, jnp.float32)
    bound = jnp.full((xq.shape[0], 1), -1.0, jnp.float32)
    cols = []
    for _ in range(K):
        cand = jnp.where(work > bound, work, jnp.inf)
        m = jnp.min(cand, axis=1, keepdims=True)
        cols.append(m)
        bound = m
    packed = jnp.concatenate(cols, axis=1)
    o_ref[0] = pltpu.bitcast(packed, jnp.int32) & jnp.int32(0xFFF)


def _knn(xyz, xq):
    B, N, _ = xyz.shape
    qt = min(QT, N)
    pts = jnp.transpose(xq, (0, 2, 1))
    idx = pl.pallas_call(
        _knn_kernel,
        out_shape=jax.ShapeDtypeStruct((B, N, K), jnp.int32),
        grid=(B, N // qt),
        in_specs=[
            pl.BlockSpec((1, qt, 8), lambda b, i: (b, i, 0)),
            pl.BlockSpec((1, 8, N), lambda b, i: (b, 0, 0)),
        ],
        out_specs=pl.BlockSpec((1, qt, K), lambda b, i: (b, i, 0)),
        compiler_params=pltpu.CompilerParams(
            dimension_semantics=("parallel", "parallel"),
            vmem_limit_bytes=100 * 1024 * 1024,
        ),
    )(xq, pts)
    return idx


def kernel(feat, xyz,
           in_fused_w, in_fused_sc, in_fused_sh,
           mlp_out_w, mlp_out_sc, mlp_out_sh,
           lse1_w, lse1_sc, lse1_sh,
           lse2_w, lse2_sc, lse2_sh,
           att1_w_aa, att1_w_ab, att1_w_ba, att1_w_bb, att1_b_a, att1_b_b,
           att2_w_aa, att2_w_ab, att2_w_ba, att2_w_bb, att2_b_a, att2_b_b,
           pool1_w_a, pool1_w_b, pool1_sc, pool1_sh,
           pool2_w_a, pool2_w_b, pool2_sc, pool2_sh):
    B, d_in, N, _ = feat.shape
    BN = B * N

    x = jnp.transpose(feat[..., 0], (0, 2, 1)).reshape(BN, d_in)

    xq_b = jnp.concatenate(
        [xyz, jnp.zeros((B, N, 5), jnp.float32)], axis=-1)
    idx = _knn(xyz, xq_b)
    xq = xq_b.reshape(BN, 8)
    idx_flat = idx.reshape(BN * K)

    def _parts(w):
        zc = jnp.zeros((5, w.shape[1]), w.dtype)
        w_c = jnp.concatenate([w[0:3] + w[6:9], zc], axis=0)
        w_n = jnp.concatenate([w[3:6] - w[6:9], zc], axis=0)
        return w_c, w_n, w[9:10]
    c1, n1, d1 = _parts(lse1_w)
    c2, n2, d2w = _parts(lse2_w)
    w_c = jnp.concatenate([c1, c2], axis=1)
    w_n = jnp.concatenate([n1, n2], axis=1)
    w_d = jnp.concatenate([d1, d2w], axis=1)
    sc_lse = jnp.concatenate([lse1_sc, lse2_sc], axis=1)
    sh_lse = jnp.concatenate([lse1_sh, lse2_sh], axis=1)
    w_att1 = jnp.concatenate([att1_w_aa, att1_w_ab], axis=1)
    w_fc1 = jnp.concatenate([att1_w_ba, att1_w_bb], axis=1)
    b1 = jnp.concatenate([att1_b_a, att1_b_b], axis=1)
    w_att2 = jnp.concatenate([att2_w_aa, att2_w_ab], axis=1)
    w_fc2 = jnp.concatenate([att2_w_ba, att2_w_bb], axis=1)
    b2 = jnp.concatenate([att2_b_a, att2_b_b], axis=1)
    w_p1 = jnp.concatenate([pool1_w_a, pool1_w_b], axis=0)
    w_p2 = jnp.concatenate([pool2_w_a, pool2_w_b], axis=0)

    nb = N // TN
    out = pl.pallas_call(
        _fused_kernel,
        out_shape=jax.ShapeDtypeStruct((BN, 256), jnp.float32),
        grid=(BN // TN,),
        in_specs=[
            pl.BlockSpec((TN, 128), lambda i: (i, 0)),
            pl.BlockSpec((TN, 8), lambda i: (i, 0)),
            pl.BlockSpec((TN * K,), lambda i: (i,),
                         memory_space=pltpu.SMEM),
            pl.BlockSpec((1, N, 8), lambda i: (i // nb, 0, 0)),
            _const_spec((128, 384)), _const_spec((1, 384)), _const_spec((1, 384)),
            _const_spec((8, 256)), _const_spec((8, 256)), _const_spec((1, 256)),
            _const_spec((1, 256)), _const_spec((1, 256)),
            _const_spec((128, 256)), _const_spec((128, 256)), _const_spec((1, 256)),
            _const_spec((256, 128)), _const_spec((1, 128)), _const_spec((1, 128)),
            _const_spec((128, 256)), _const_spec((128, 256)), _const_spec((1, 256)),
            _const_spec((256, 128)), _const_spec((1, 128)), _const_spec((1, 128)),
            _const_spec((128, 256)), _const_spec((1, 256)), _const_spec((1, 256)),
        ],
        out_specs=pl.BlockSpec((TN, 256), lambda i: (i, 0)),
        scratch_shapes=[pltpu.VMEM((TN * K, 8), jnp.float32)],
        compiler_params=pltpu.CompilerParams(
            dimension_semantics=("parallel",),
            vmem_limit_bytes=100 * 1024 * 1024,
        ),
    )(x, xq, idx_flat, xq_b.reshape(B, N, 8),
      in_fused_w, in_fused_sc, in_fused_sh,
      w_c, w_n, w_d, sc_lse, sh_lse,
      w_att1, w_fc1, b1, w_p1, pool1_sc, pool1_sh,
      w_att2, w_fc2, b2, w_p2, pool2_sc, pool2_sh,
      mlp_out_w, mlp_out_sc, mlp_out_sh)

    out = out.reshape(B, N, 256)
    return jnp.transpose(out, (0, 2, 1))[..., None]


# speedup vs baseline: 2.4009x; 1.0244x over previous
import jax
import jax.numpy as jnp
from jax.experimental import pallas as pl
from jax.experimental.pallas import tpu as pltpu

K = 16
TN = 128


def _softmax_k(lg):
    m = jnp.max(lg, axis=1, keepdims=True)
    e = jnp.exp(lg - m)
    return e * pl.reciprocal(jnp.sum(e, axis=1, keepdims=True), approx=True)


def _fused_kernel(x_ref, xq_ref, idx_ref, ptsb_ref,
                  w_if_ref, sc_if_ref, sh_if_ref,
                  w_c_ref, w_n_ref, w_d_ref, sc_lse_ref, sh_lse_ref,
                  w_att1_ref, w_fc1_ref, b1_ref,
                  w_p1_ref, sc_p1_ref, sh_p1_ref,
                  w_att2_ref, w_fc2_ref, b2_ref,
                  w_p2_ref, sc_p2_ref, sh_p2_ref,
                  w_mo_ref, sc_mo_ref, sh_mo_ref,
                  o_ref, gath_ref):
    tn = x_ref.shape[0]

    def body(i, _):
        gath_ref[i, :] = ptsb_ref[0, idx_ref[i], :]
        return 0
    jax.lax.fori_loop(0, tn * K, body, 0, unroll=32)

    y = jnp.dot(x_ref[...], w_if_ref[...], preferred_element_type=jnp.float32)
    y = jnp.maximum(y * sc_if_ref[...] + sh_if_ref[...], 0.0)
    residual = y[:, :256]
    fc1 = y[:, 256:384]

    g = gath_ref[...]
    xq = xq_ref[...]
    diff = g.reshape(tn, K, 8) - xq[:, None, :]
    d2 = jnp.sum(diff * diff, axis=2, keepdims=True)
    cproj = jnp.dot(xq, w_c_ref[...], preferred_element_type=jnp.float32)
    nproj = jnp.dot(g, w_n_ref[...], preferred_element_type=jnp.float32)
    lse = nproj.reshape(tn, K, 256) + cproj[:, None, :] + d2 * w_d_ref[0][None, None, :]
    lse = jnp.maximum(lse * sc_lse_ref[0][None, None, :]
                      + sh_lse_ref[0][None, None, :], 0.0)
    lsef = lse.reshape(tn * K, 256)
    lse1f = lsef[:, :128]
    lse2f = lsef[:, 128:]

    def attend(lse_flat, fc, w_att_ref, w_fc_ref, b_ref,
               w_p_ref, sc_p_ref, sh_p_ref):
        f = jnp.dot(fc, w_fc_ref[...],
                    preferred_element_type=jnp.float32) + b_ref[...]
        lg = jnp.dot(lse_flat, w_att_ref[...],
                     preferred_element_type=jnp.float32)
        lg = lg.reshape(tn, K, 256) + f[:, None, :]
        m = jnp.max(lg, axis=1, keepdims=True)
        e = jnp.exp(lg - m)
        s = e * pl.reciprocal(jnp.sum(e, axis=1, keepdims=True), approx=True)
        agg_a = jnp.sum(s[:, :, :128] * lse_flat.reshape(tn, K, 128), axis=1)
        agg_b = fc * jnp.sum(s[:, :, 128:], axis=1)
        p = jnp.dot(jnp.concatenate([agg_a, agg_b], axis=1), w_p_ref[...],
                    preferred_element_type=jnp.float32)
        return jnp.maximum(p * sc_p_ref[...] + sh_p_ref[...], 0.0)

    att1 = attend(lse1f, fc1, w_att1_ref, w_fc1_ref, b1_ref,
                  w_p1_ref, sc_p1_ref, sh_p1_ref)
    att2 = attend(lse2f, att1, w_att2_ref, w_fc2_ref, b2_ref,
                  w_p2_ref, sc_p2_ref, sh_p2_ref)

    z = jnp.dot(att2, w_mo_ref[...], preferred_element_type=jnp.float32)
    z = jnp.maximum(z * sc_mo_ref[...] + sh_mo_ref[...], 0.0) + residual
    o_ref[...] = jnp.where(z >= 0.0, z, 0.01 * z)


def _const_spec(shape):
    return pl.BlockSpec(shape, lambda *_: (0,) * len(shape))


QT = 512


def _knn_kernel(xq_ref, pts_ref, o_ref):
    xq = xq_ref[0]
    px = pts_ref[0, 0:1, :]
    py = pts_ref[0, 1:2, :]
    pz = pts_ref[0, 2:3, :]
    dx = xq[:, 0:1] - px
    dy = xq[:, 1:2] - py
    dz = xq[:, 2:3] - pz
    d2 = dx * dx + dy * dy + dz * dz
    lane = jax.lax.broadcasted_iota(jnp.int32, d2.shape, 1)
    key = ((pltpu.bitcast(d2, jnp.int32) & ~jnp.int32(0xFFF)) | lane) \
        + jnp.int32(0x20000000)
    work = pltpu.bitcast(key, jnp.float32)
    bound = jnp.full((xq.shape[0], 1), -1.0, jnp.float32)
    cols = []
    for _ in range(K):
        cand = jnp.where(work > bound, work, jnp.inf)
        m = jnp.min(cand, axis=1, keepdims=True)
        cols.append(m)
        bound = m
    packed = jnp.concatenate(cols, axis=1)
    o_ref[0] = pltpu.bitcast(packed, jnp.int32) & jnp.int32(0xFFF)


def _knn(xyz, xq):
    B, N, _ = xyz.shape
    qt = min(QT, N)
    pts = jnp.transpose(xq, (0, 2, 1))
    idx = pl.pallas_call(
        _knn_kernel,
        out_shape=jax.ShapeDtypeStruct((B, N, K), jnp.int32),
        grid=(B, N // qt),
        in_specs=[
            pl.BlockSpec((1, qt, 8), lambda b, i: (b, i, 0)),
            pl.BlockSpec((1, 8, N), lambda b, i: (b, 0, 0)),
        ],
        out_specs=pl.BlockSpec((1, qt, K), lambda b, i: (b, i, 0)),
        compiler_params=pltpu.CompilerParams(
            dimension_semantics=("parallel", "parallel"),
            vmem_limit_bytes=100 * 1024 * 1024,
        ),
    )(xq, pts)
    return idx


def kernel(feat, xyz,
           in_fused_w, in_fused_sc, in_fused_sh,
           mlp_out_w, mlp_out_sc, mlp_out_sh,
           lse1_w, lse1_sc, lse1_sh,
           lse2_w, lse2_sc, lse2_sh,
           att1_w_aa, att1_w_ab, att1_w_ba, att1_w_bb, att1_b_a, att1_b_b,
           att2_w_aa, att2_w_ab, att2_w_ba, att2_w_bb, att2_b_a, att2_b_b,
           pool1_w_a, pool1_w_b, pool1_sc, pool1_sh,
           pool2_w_a, pool2_w_b, pool2_sc, pool2_sh):
    B, d_in, N, _ = feat.shape
    BN = B * N

    x = jnp.transpose(feat[..., 0], (0, 2, 1)).reshape(BN, d_in)

    xq_b = jnp.concatenate(
        [xyz, jnp.zeros((B, N, 5), jnp.float32)], axis=-1)
    idx = _knn(xyz, xq_b)
    xq = xq_b.reshape(BN, 8)
    idx_flat = idx.reshape(BN * K)

    def _parts(w):
        zc = jnp.zeros((5, w.shape[1]), w.dtype)
        w_c = jnp.concatenate([w[0:3] + w[6:9], zc], axis=0)
        w_n = jnp.concatenate([w[3:6] - w[6:9], zc], axis=0)
        return w_c, w_n, w[9:10]
    c1, n1, d1 = _parts(lse1_w)
    c2, n2, d2w = _parts(lse2_w)
    w_c = jnp.concatenate([c1, c2], axis=1)
    w_n = jnp.concatenate([n1, n2], axis=1)
    w_d = jnp.concatenate([d1, d2w], axis=1)
    sc_lse = jnp.concatenate([lse1_sc, lse2_sc], axis=1)
    sh_lse = jnp.concatenate([lse1_sh, lse2_sh], axis=1)
    w_att1 = jnp.concatenate([att1_w_aa, att1_w_ab], axis=1)
    w_fc1 = jnp.concatenate([att1_w_ba, att1_w_bb], axis=1)
    b1 = jnp.concatenate([att1_b_a, att1_b_b], axis=1)
    w_att2 = jnp.concatenate([att2_w_aa, att2_w_ab], axis=1)
    w_fc2 = jnp.concatenate([att2_w_ba, att2_w_bb], axis=1)
    b2 = jnp.concatenate([att2_b_a, att2_b_b], axis=1)
    w_p1 = jnp.concatenate([pool1_w_a, pool1_w_b], axis=0)
    w_p2 = jnp.concatenate([pool2_w_a, pool2_w_b], axis=0)

    nb = N // TN
    out = pl.pallas_call(
        _fused_kernel,
        out_shape=jax.ShapeDtypeStruct((BN, 256), jnp.float32),
        grid=(BN // TN,),
        in_specs=[
            pl.BlockSpec((TN, 128), lambda i: (i, 0)),
            pl.BlockSpec((TN, 8), lambda i: (i, 0)),
            pl.BlockSpec((TN * K,), lambda i: (i,),
                         memory_space=pltpu.SMEM),
            pl.BlockSpec((1, N, 8), lambda i: (i // nb, 0, 0)),
            _const_spec((128, 384)), _const_spec((1, 384)), _const_spec((1, 384)),
            _const_spec((8, 256)), _const_spec((8, 256)), _const_spec((1, 256)),
            _const_spec((1, 256)), _const_spec((1, 256)),
            _const_spec((128, 256)), _const_spec((128, 256)), _const_spec((1, 256)),
            _const_spec((256, 128)), _const_spec((1, 128)), _const_spec((1, 128)),
            _const_spec((128, 256)), _const_spec((128, 256)), _const_spec((1, 256)),
            _const_spec((256, 128)), _const_spec((1, 128)), _const_spec((1, 128)),
            _const_spec((128, 256)), _const_spec((1, 256)), _const_spec((1, 256)),
        ],
        out_specs=pl.BlockSpec((TN, 256), lambda i: (i, 0)),
        scratch_shapes=[pltpu.VMEM((TN * K, 8), jnp.float32)],
        compiler_params=pltpu.CompilerParams(
            dimension_semantics=("parallel",),
            vmem_limit_bytes=100 * 1024 * 1024,
        ),
    )(x, xq, idx_flat, xq_b.reshape(B, N, 8),
      in_fused_w, in_fused_sc, in_fused_sh,
      w_c, w_n, w_d, sc_lse, sh_lse,
      w_att1, w_fc1, b1, w_p1, pool1_sc, pool1_sh,
      w_att2, w_fc2, b2, w_p2, pool2_sc, pool2_sh,
      mlp_out_w, mlp_out_sc, mlp_out_sh)

    out = out.reshape(B, N, 256)
    return jnp.transpose(out, (0, 2, 1))[..., None]


# speedup vs baseline: 2.4573x; 1.0235x over previous
import jax
import jax.numpy as jnp
from jax.experimental import pallas as pl
from jax.experimental.pallas import tpu as pltpu

K = 16
TN = 256


def _softmax_k(lg):
    m = jnp.max(lg, axis=1, keepdims=True)
    e = jnp.exp(lg - m)
    return e * pl.reciprocal(jnp.sum(e, axis=1, keepdims=True), approx=True)


def _fused_kernel(x_ref, xq_ref, idx_ref, ptsb_ref,
                  w_if_ref, sc_if_ref, sh_if_ref,
                  w_c_ref, w_n_ref, w_d_ref, sc_lse_ref, sh_lse_ref,
                  w_att1_ref, w_fc1_ref, b1_ref,
                  w_p1_ref, sc_p1_ref, sh_p1_ref,
                  w_att2_ref, w_fc2_ref, b2_ref,
                  w_p2_ref, sc_p2_ref, sh_p2_ref,
                  w_mo_ref, sc_mo_ref, sh_mo_ref,
                  o_ref, gath_ref):
    tn = x_ref.shape[0]

    def body(i, _):
        gath_ref[i, :] = ptsb_ref[0, idx_ref[i], :]
        return 0
    jax.lax.fori_loop(0, tn * K, body, 0, unroll=32)

    y = jnp.dot(x_ref[...], w_if_ref[...], preferred_element_type=jnp.float32)
    y = jnp.maximum(y * sc_if_ref[...] + sh_if_ref[...], 0.0)
    residual = y[:, :256]
    fc1 = y[:, 256:384]

    g = gath_ref[...]
    xq = xq_ref[...]
    diff = g.reshape(tn, K, 8) - xq[:, None, :]
    d2 = jnp.sum(diff * diff, axis=2, keepdims=True)
    cproj = jnp.dot(xq, w_c_ref[...], preferred_element_type=jnp.float32)
    nproj = jnp.dot(g, w_n_ref[...], preferred_element_type=jnp.float32)
    lse = nproj.reshape(tn, K, 256) + cproj[:, None, :] + d2 * w_d_ref[0][None, None, :]
    lse = jnp.maximum(lse * sc_lse_ref[0][None, None, :]
                      + sh_lse_ref[0][None, None, :], 0.0)
    lsef = lse.reshape(tn * K, 256)
    lse1f = lsef[:, :128]
    lse2f = lsef[:, 128:]

    def attend(lse_flat, fc, w_att_ref, w_fc_ref, b_ref,
               w_p_ref, sc_p_ref, sh_p_ref):
        f = jnp.dot(fc, w_fc_ref[...],
                    preferred_element_type=jnp.float32) + b_ref[...]
        lg = jnp.dot(lse_flat, w_att_ref[...],
                     preferred_element_type=jnp.float32)
        lg = lg.reshape(tn, K, 256) + f[:, None, :]
        m = jnp.max(lg, axis=1, keepdims=True)
        e = jnp.exp(lg - m)
        s = e * pl.reciprocal(jnp.sum(e, axis=1, keepdims=True), approx=True)
        agg_a = jnp.sum(s[:, :, :128] * lse_flat.reshape(tn, K, 128), axis=1)
        agg_b = fc * jnp.sum(s[:, :, 128:], axis=1)
        p = jnp.dot(jnp.concatenate([agg_a, agg_b], axis=1), w_p_ref[...],
                    preferred_element_type=jnp.float32)
        return jnp.maximum(p * sc_p_ref[...] + sh_p_ref[...], 0.0)

    att1 = attend(lse1f, fc1, w_att1_ref, w_fc1_ref, b1_ref,
                  w_p1_ref, sc_p1_ref, sh_p1_ref)
    att2 = attend(lse2f, att1, w_att2_ref, w_fc2_ref, b2_ref,
                  w_p2_ref, sc_p2_ref, sh_p2_ref)

    z = jnp.dot(att2, w_mo_ref[...], preferred_element_type=jnp.float32)
    z = jnp.maximum(z * sc_mo_ref[...] + sh_mo_ref[...], 0.0) + residual
    o_ref[...] = jnp.where(z >= 0.0, z, 0.01 * z)


def _const_spec(shape):
    return pl.BlockSpec(shape, lambda *_: (0,) * len(shape))


QT = 512


def _knn_kernel(xq_ref, pts_ref, o_ref):
    xq = xq_ref[0]
    px = pts_ref[0, 0:1, :]
    py = pts_ref[0, 1:2, :]
    pz = pts_ref[0, 2:3, :]
    dx = xq[:, 0:1] - px
    dy = xq[:, 1:2] - py
    dz = xq[:, 2:3] - pz
    d2 = dx * dx + dy * dy + dz * dz
    lane = jax.lax.broadcasted_iota(jnp.int32, d2.shape, 1)
    key = ((pltpu.bitcast(d2, jnp.int32) & ~jnp.int32(0xFFF)) | lane) \
        + jnp.int32(0x20000000)
    work = pltpu.bitcast(key, jnp.float32)
    bound = jnp.full((xq.shape[0], 1), -1.0, jnp.float32)
    cols = []
    for _ in range(K):
        cand = jnp.where(work > bound, work, jnp.inf)
        m = jnp.min(cand, axis=1, keepdims=True)
        cols.append(m)
        bound = m
    packed = jnp.concatenate(cols, axis=1)
    o_ref[0] = pltpu.bitcast(packed, jnp.int32) & jnp.int32(0xFFF)


def _knn(xyz, xq):
    B, N, _ = xyz.shape
    qt = min(QT, N)
    pts = jnp.transpose(xq, (0, 2, 1))
    idx = pl.pallas_call(
        _knn_kernel,
        out_shape=jax.ShapeDtypeStruct((B, N, K), jnp.int32),
        grid=(B, N // qt),
        in_specs=[
            pl.BlockSpec((1, qt, 8), lambda b, i: (b, i, 0)),
            pl.BlockSpec((1, 8, N), lambda b, i: (b, 0, 0)),
        ],
        out_specs=pl.BlockSpec((1, qt, K), lambda b, i: (b, i, 0)),
        compiler_params=pltpu.CompilerParams(
            dimension_semantics=("parallel", "parallel"),
            vmem_limit_bytes=100 * 1024 * 1024,
        ),
    )(xq, pts)
    return idx


def kernel(feat, xyz,
           in_fused_w, in_fused_sc, in_fused_sh,
           mlp_out_w, mlp_out_sc, mlp_out_sh,
           lse1_w, lse1_sc, lse1_sh,
           lse2_w, lse2_sc, lse2_sh,
           att1_w_aa, att1_w_ab, att1_w_ba, att1_w_bb, att1_b_a, att1_b_b,
           att2_w_aa, att2_w_ab, att2_w_ba, att2_w_bb, att2_b_a, att2_b_b,
           pool1_w_a, pool1_w_b, pool1_sc, pool1_sh,
           pool2_w_a, pool2_w_b, pool2_sc, pool2_sh):
    B, d_in, N, _ = feat.shape
    BN = B * N

    x = jnp.transpose(feat[..., 0], (0, 2, 1)).reshape(BN, d_in)

    xq_b = jnp.concatenate(
        [xyz, jnp.zeros((B, N, 5), jnp.float32)], axis=-1)
    idx = _knn(xyz, xq_b)
    xq = xq_b.reshape(BN, 8)
    idx_flat = idx.reshape(BN * K)

    def _parts(w):
        zc = jnp.zeros((5, w.shape[1]), w.dtype)
        w_c = jnp.concatenate([w[0:3] + w[6:9], zc], axis=0)
        w_n = jnp.concatenate([w[3:6] - w[6:9], zc], axis=0)
        return w_c, w_n, w[9:10]
    c1, n1, d1 = _parts(lse1_w)
    c2, n2, d2w = _parts(lse2_w)
    w_c = jnp.concatenate([c1, c2], axis=1)
    w_n = jnp.concatenate([n1, n2], axis=1)
    w_d = jnp.concatenate([d1, d2w], axis=1)
    sc_lse = jnp.concatenate([lse1_sc, lse2_sc], axis=1)
    sh_lse = jnp.concatenate([lse1_sh, lse2_sh], axis=1)
    w_att1 = jnp.concatenate([att1_w_aa, att1_w_ab], axis=1)
    w_fc1 = jnp.concatenate([att1_w_ba, att1_w_bb], axis=1)
    b1 = jnp.concatenate([att1_b_a, att1_b_b], axis=1)
    w_att2 = jnp.concatenate([att2_w_aa, att2_w_ab], axis=1)
    w_fc2 = jnp.concatenate([att2_w_ba, att2_w_bb], axis=1)
    b2 = jnp.concatenate([att2_b_a, att2_b_b], axis=1)
    w_p1 = jnp.concatenate([pool1_w_a, pool1_w_b], axis=0)
    w_p2 = jnp.concatenate([pool2_w_a, pool2_w_b], axis=0)

    nb = N // TN
    out = pl.pallas_call(
        _fused_kernel,
        out_shape=jax.ShapeDtypeStruct((BN, 256), jnp.float32),
        grid=(BN // TN,),
        in_specs=[
            pl.BlockSpec((TN, 128), lambda i: (i, 0)),
            pl.BlockSpec((TN, 8), lambda i: (i, 0)),
            pl.BlockSpec((TN * K,), lambda i: (i,),
                         memory_space=pltpu.SMEM),
            pl.BlockSpec((1, N, 8), lambda i: (i // nb, 0, 0)),
            _const_spec((128, 384)), _const_spec((1, 384)), _const_spec((1, 384)),
            _const_spec((8, 256)), _const_spec((8, 256)), _const_spec((1, 256)),
            _const_spec((1, 256)), _const_spec((1, 256)),
            _const_spec((128, 256)), _const_spec((128, 256)), _const_spec((1, 256)),
            _const_spec((256, 128)), _const_spec((1, 128)), _const_spec((1, 128)),
            _const_spec((128, 256)), _const_spec((128, 256)), _const_spec((1, 256)),
            _const_spec((256, 128)), _const_spec((1, 128)), _const_spec((1, 128)),
            _const_spec((128, 256)), _const_spec((1, 256)), _const_spec((1, 256)),
        ],
        out_specs=pl.BlockSpec((TN, 256), lambda i: (i, 0)),
        scratch_shapes=[pltpu.VMEM((TN * K, 8), jnp.float32)],
        compiler_params=pltpu.CompilerParams(
            dimension_semantics=("parallel",),
            vmem_limit_bytes=100 * 1024 * 1024,
        ),
    )(x, xq, idx_flat, xq_b.reshape(B, N, 8),
      in_fused_w, in_fused_sc, in_fused_sh,
      w_c, w_n, w_d, sc_lse, sh_lse,
      w_att1, w_fc1, b1, w_p1, pool1_sc, pool1_sh,
      w_att2, w_fc2, b2, w_p2, pool2_sc, pool2_sh,
      mlp_out_w, mlp_out_sc, mlp_out_sh)

    out = out.reshape(B, N, 256)
    return jnp.transpose(out, (0, 2, 1))[..., None]


# speedup vs baseline: 2.4810x; 1.0096x over previous
import jax
import jax.numpy as jnp
from jax.experimental import pallas as pl
from jax.experimental.pallas import tpu as pltpu

K = 16
TN = 512


def _softmax_k(lg):
    m = jnp.max(lg, axis=1, keepdims=True)
    e = jnp.exp(lg - m)
    return e * pl.reciprocal(jnp.sum(e, axis=1, keepdims=True), approx=True)


def _fused_kernel(x_ref, xq_ref, idx_ref, ptsb_ref,
                  w_if_ref, sc_if_ref, sh_if_ref,
                  w_c_ref, w_n_ref, w_d_ref, sc_lse_ref, sh_lse_ref,
                  w_att1_ref, w_fc1_ref, b1_ref,
                  w_p1_ref, sc_p1_ref, sh_p1_ref,
                  w_att2_ref, w_fc2_ref, b2_ref,
                  w_p2_ref, sc_p2_ref, sh_p2_ref,
                  w_mo_ref, sc_mo_ref, sh_mo_ref,
                  o_ref, gath_ref):
    tn = x_ref.shape[0]

    def body(i, _):
        gath_ref[i, :] = ptsb_ref[0, idx_ref[i], :]
        return 0
    jax.lax.fori_loop(0, tn * K, body, 0, unroll=32)

    y = jnp.dot(x_ref[...], w_if_ref[...], preferred_element_type=jnp.float32)
    y = jnp.maximum(y * sc_if_ref[...] + sh_if_ref[...], 0.0)
    residual = y[:, :256]
    fc1 = y[:, 256:384]

    g = gath_ref[...]
    xq = xq_ref[...]
    diff = g.reshape(tn, K, 8) - xq[:, None, :]
    d2 = jnp.sum(diff * diff, axis=2, keepdims=True)
    cproj = jnp.dot(xq, w_c_ref[...], preferred_element_type=jnp.float32)
    nproj = jnp.dot(g, w_n_ref[...], preferred_element_type=jnp.float32)
    lse = nproj.reshape(tn, K, 256) + cproj[:, None, :] + d2 * w_d_ref[0][None, None, :]
    lse = jnp.maximum(lse * sc_lse_ref[0][None, None, :]
                      + sh_lse_ref[0][None, None, :], 0.0)
    lsef = lse.reshape(tn * K, 256)
    lse1f = lsef[:, :128]
    lse2f = lsef[:, 128:]

    def attend(lse_flat, fc, w_att_ref, w_fc_ref, b_ref,
               w_p_ref, sc_p_ref, sh_p_ref):
        f = jnp.dot(fc, w_fc_ref[...],
                    preferred_element_type=jnp.float32) + b_ref[...]
        lg = jnp.dot(lse_flat, w_att_ref[...],
                     preferred_element_type=jnp.float32)
        lg = lg.reshape(tn, K, 256) + f[:, None, :]
        m = jnp.max(lg, axis=1, keepdims=True)
        e = jnp.exp(lg - m)
        s = e * pl.reciprocal(jnp.sum(e, axis=1, keepdims=True), approx=True)
        agg_a = jnp.sum(s[:, :, :128] * lse_flat.reshape(tn, K, 128), axis=1)
        agg_b = fc * jnp.sum(s[:, :, 128:], axis=1)
        p = jnp.dot(jnp.concatenate([agg_a, agg_b], axis=1), w_p_ref[...],
                    preferred_element_type=jnp.float32)
        return jnp.maximum(p * sc_p_ref[...] + sh_p_ref[...], 0.0)

    att1 = attend(lse1f, fc1, w_att1_ref, w_fc1_ref, b1_ref,
                  w_p1_ref, sc_p1_ref, sh_p1_ref)
    att2 = attend(lse2f, att1, w_att2_ref, w_fc2_ref, b2_ref,
                  w_p2_ref, sc_p2_ref, sh_p2_ref)

    z = jnp.dot(att2, w_mo_ref[...], preferred_element_type=jnp.float32)
    z = jnp.maximum(z * sc_mo_ref[...] + sh_mo_ref[...], 0.0) + residual
    o_ref[...] = jnp.where(z >= 0.0, z, 0.01 * z)


def _const_spec(shape):
    return pl.BlockSpec(shape, lambda *_: (0,) * len(shape))


QT = 512


def _knn_kernel(xq_ref, pts_ref, o_ref):
    xq = xq_ref[0]
    px = pts_ref[0, 0:1, :]
    py = pts_ref[0, 1:2, :]
    pz = pts_ref[0, 2:3, :]
    dx = xq[:, 0:1] - px
    dy = xq[:, 1:2] - py
    dz = xq[:, 2:3] - pz
    d2 = dx * dx + dy * dy + dz * dz
    lane = jax.lax.broadcasted_iota(jnp.int32, d2.shape, 1)
    key = ((pltpu.bitcast(d2, jnp.int32) & ~jnp.int32(0xFFF)) | lane) \
        + jnp.int32(0x20000000)
    work = pltpu.bitcast(key, jnp.float32)
    bound = jnp.full((xq.shape[0], 1), -1.0, jnp.float32)
    cols = []
    for _ in range(K):
        cand = jnp.where(work > bound, work, jnp.inf)
        m = jnp.min(cand, axis=1, keepdims=True)
        cols.append(m)
        bound = m
    packed = jnp.concatenate(cols, axis=1)
    o_ref[0] = pltpu.bitcast(packed, jnp.int32) & jnp.int32(0xFFF)


def _knn(xyz, xq):
    B, N, _ = xyz.shape
    qt = min(QT, N)
    pts = jnp.transpose(xq, (0, 2, 1))
    idx = pl.pallas_call(
        _knn_kernel,
        out_shape=jax.ShapeDtypeStruct((B, N, K), jnp.int32),
        grid=(B, N // qt),
        in_specs=[
            pl.BlockSpec((1, qt, 8), lambda b, i: (b, i, 0)),
            pl.BlockSpec((1, 8, N), lambda b, i: (b, 0, 0)),
        ],
        out_specs=pl.BlockSpec((1, qt, K), lambda b, i: (b, i, 0)),
        compiler_params=pltpu.CompilerParams(
            dimension_semantics=("parallel", "parallel"),
            vmem_limit_bytes=100 * 1024 * 1024,
        ),
    )(xq, pts)
    return idx


def kernel(feat, xyz,
           in_fused_w, in_fused_sc, in_fused_sh,
           mlp_out_w, mlp_out_sc, mlp_out_sh,
           lse1_w, lse1_sc, lse1_sh,
           lse2_w, lse2_sc, lse2_sh,
           att1_w_aa, att1_w_ab, att1_w_ba, att1_w_bb, att1_b_a, att1_b_b,
           att2_w_aa, att2_w_ab, att2_w_ba, att2_w_bb, att2_b_a, att2_b_b,
           pool1_w_a, pool1_w_b, pool1_sc, pool1_sh,
           pool2_w_a, pool2_w_b, pool2_sc, pool2_sh):
    B, d_in, N, _ = feat.shape
    BN = B * N

    x = jnp.transpose(feat[..., 0], (0, 2, 1)).reshape(BN, d_in)

    xq_b = jnp.concatenate(
        [xyz, jnp.zeros((B, N, 5), jnp.float32)], axis=-1)
    idx = _knn(xyz, xq_b)
    xq = xq_b.reshape(BN, 8)
    idx_flat = idx.reshape(BN * K)

    def _parts(w):
        zc = jnp.zeros((5, w.shape[1]), w.dtype)
        w_c = jnp.concatenate([w[0:3] + w[6:9], zc], axis=0)
        w_n = jnp.concatenate([w[3:6] - w[6:9], zc], axis=0)
        return w_c, w_n, w[9:10]
    c1, n1, d1 = _parts(lse1_w)
    c2, n2, d2w = _parts(lse2_w)
    w_c = jnp.concatenate([c1, c2], axis=1)
    w_n = jnp.concatenate([n1, n2], axis=1)
    w_d = jnp.concatenate([d1, d2w], axis=1)
    sc_lse = jnp.concatenate([lse1_sc, lse2_sc], axis=1)
    sh_lse = jnp.concatenate([lse1_sh, lse2_sh], axis=1)
    w_att1 = jnp.concatenate([att1_w_aa, att1_w_ab], axis=1)
    w_fc1 = jnp.concatenate([att1_w_ba, att1_w_bb], axis=1)
    b1 = jnp.concatenate([att1_b_a, att1_b_b], axis=1)
    w_att2 = jnp.concatenate([att2_w_aa, att2_w_ab], axis=1)
    w_fc2 = jnp.concatenate([att2_w_ba, att2_w_bb], axis=1)
    b2 = jnp.concatenate([att2_b_a, att2_b_b], axis=1)
    w_p1 = jnp.concatenate([pool1_w_a, pool1_w_b], axis=0)
    w_p2 = jnp.concatenate([pool2_w_a, pool2_w_b], axis=0)

    nb = N // TN
    out = pl.pallas_call(
        _fused_kernel,
        out_shape=jax.ShapeDtypeStruct((BN, 256), jnp.float32),
        grid=(BN // TN,),
        in_specs=[
            pl.BlockSpec((TN, 128), lambda i: (i, 0)),
            pl.BlockSpec((TN, 8), lambda i: (i, 0)),
            pl.BlockSpec((TN * K,), lambda i: (i,),
                         memory_space=pltpu.SMEM),
            pl.BlockSpec((1, N, 8), lambda i: (i // nb, 0, 0)),
            _const_spec((128, 384)), _const_spec((1, 384)), _const_spec((1, 384)),
            _const_spec((8, 256)), _const_spec((8, 256)), _const_spec((1, 256)),
            _const_spec((1, 256)), _const_spec((1, 256)),
            _const_spec((128, 256)), _const_spec((128, 256)), _const_spec((1, 256)),
            _const_spec((256, 128)), _const_spec((1, 128)), _const_spec((1, 128)),
            _const_spec((128, 256)), _const_spec((128, 256)), _const_spec((1, 256)),
            _const_spec((256, 128)), _const_spec((1, 128)), _const_spec((1, 128)),
            _const_spec((128, 256)), _const_spec((1, 256)), _const_spec((1, 256)),
        ],
        out_specs=pl.BlockSpec((TN, 256), lambda i: (i, 0)),
        scratch_shapes=[pltpu.VMEM((TN * K, 8), jnp.float32)],
        compiler_params=pltpu.CompilerParams(
            dimension_semantics=("parallel",),
            vmem_limit_bytes=100 * 1024 * 1024,
        ),
    )(x, xq, idx_flat, xq_b.reshape(B, N, 8),
      in_fused_w, in_fused_sc, in_fused_sh,
      w_c, w_n, w_d, sc_lse, sh_lse,
      w_att1, w_fc1, b1, w_p1, pool1_sc, pool1_sh,
      w_att2, w_fc2, b2, w_p2, pool2_sc, pool2_sh,
      mlp_out_w, mlp_out_sc, mlp_out_sh)

    out = out.reshape(B, N, 256)
    return jnp.transpose(out, (0, 2, 1))[..., None]
